# Initial kernel scaffold; baseline (speedup 1.0000x reference)
import jax
import jax.numpy as jnp
from jax.experimental import pallas as pl


def kernel(hidden, eps, frac_coords, gt_lengths_u, gt_angles_u, cart_noise, W_mu, b_mu, W_var, b_var, Wn1, bn1, Wn2, bn2, Wl1, bl1, Wl2, bl2, Wc1, bc1, Wc2, bc2, type_emb, Wd1, bd1, Wd2, bd2, Wa, ba, scaler_mean, scaler_std, sigmas, type_sigmas, batch_idx, atom_types, num_atoms_t, noise_level, type_noise_level):
    raise NotImplementedError("write your pallas kernel here")


# speedup vs baseline: 5.8554x; 5.8554x over previous
import functools
import jax
import jax.numpy as jnp
from jax import lax
from jax.experimental import pallas as pl
from jax.experimental.pallas import tpu as pltpu
from jax.experimental.pallas import tpu_sc as plsc

B = 4096
N = 81920
D = 256
A = 100
NNL = 50
NC, NS = 2, 16
NW = NC * NS
APW = N // NW
CH = 128
NCHUNK = APW // CH
CH1 = 64
NCH1 = APW // CH1
BB = 512
BBM = 1024
NB = 1024

_f32 = jnp.float32
_i32 = jnp.int32


def _tca_body(hid, eps, wmu, bmu, wvar, bvar, wn1, bn1, wn2p, bn2p,
              wl1, bl1, wl2p, bl2p, wc1, bc1, wc2, bc2, wd1, bd1, temb,
              nat, glu, gau, nl, tnl, sigt, tsigt, sm6, ss6,
              z_o, zw_o, comp_o, logp_o, pts_o, crys_o, pnum_o, plap_o,
              lmse_o, kld_o, tw_o):
    mu = hid[...] @ wmu[...] + bmu[...]
    lv = hid[...] @ wvar[...] + bvar[...]
    z = eps[...] * jnp.exp(0.5 * lv) + mu
    z_o[...] = z
    kld_o[...] = jnp.sum(-0.5 * (1.0 + lv - mu * mu - jnp.exp(lv)),
                         axis=1, keepdims=True)
    pnum_o[...] = jnp.maximum(z @ wn1[...] + bn1[...], 0.0) @ wn2p[...] + bn2p[...]
    pla = jnp.maximum(z @ wl1[...] + bl1[...], 0.0) @ wl2p[...] + bl2p[...]
    pla6 = pla[:, :6]
    naf = (nat[...] + 1).astype(_f32)
    cbrt = jnp.exp(jnp.log(naf) / 3.0)
    scaled = pla6 * ss6[...] + sm6[...]
    plen = scaled[:, :3] * cbrt
    pang = scaled[:, 3:6]
    zpad4 = jnp.zeros_like(pla6[:, :4])
    plap_o[...] = jnp.concatenate([pla6, plen, pang, zpad4], axis=1)
    gl = 2.0 + 8.0 * glu[...]
    ga = 60.0 + 60.0 * gau[...]
    tla = jnp.concatenate([gl / cbrt, ga], axis=1)
    ts = (tla - sm6[...]) / ss6[...]
    dmse = pla6 - ts
    lmse_o[...] = jnp.sum(dmse * dmse, axis=1, keepdims=True)
    compc = jnp.maximum(z @ wc1[...] + bc1[...], 0.0) @ wc2[...] + bc2[...]
    comp_o[...] = jnp.concatenate([compc, jnp.zeros_like(compc[:, :4])], axis=1)
    mx = jnp.max(compc, axis=1, keepdims=True)
    e = jnp.exp(compc - mx)
    se = jnp.sum(e, axis=1, keepdims=True)
    logp = compc - mx - jnp.log(se)
    probs = e / se
    pad28 = jnp.zeros_like(compc[:, :28])
    logp_o[...] = jnp.concatenate([logp, pad28], axis=1)
    i64 = lax.broadcasted_iota(_i32, (nl.shape[0], 64), 1)
    sig = jnp.sum(jnp.where(i64 == nl[...], sigt[...], 0.0), axis=1)
    tsig = jnp.sum(jnp.where(i64 == tnl[...], tsigt[...], 0.0), axis=1)
    pts = probs * tsig[:, None]
    pts_o[...] = jnp.concatenate([pts, pad28], axis=1)
    m1 = jnp.max(pts, axis=1)
    i100 = lax.broadcasted_iota(_i32, pts.shape, 1)
    j1 = jnp.min(jnp.where(pts == m1[:, None], i100, 10 ** 6), axis=1)
    ang = ga * (jnp.pi / 180.0)
    cc = jnp.cos(ang)
    ss_ = jnp.sin(ang)
    val = jnp.clip((cc[:, 0] * cc[:, 1] - cc[:, 2]) / (ss_[:, 0] * ss_[:, 1]),
                   -1.0 + 1e-6, 1.0 - 1e-6)
    cg = val
    sg = jnp.sqrt(1.0 - val * val)
    zc = jnp.zeros_like(val)
    m00 = gl[:, 0] * ss_[:, 1]
    m01 = zc
    m02 = gl[:, 0] * cc[:, 1]
    m10 = -gl[:, 1] * ss_[:, 0] * cg
    m11 = gl[:, 1] * ss_[:, 0] * sg
    m12 = gl[:, 1] * cc[:, 0]
    m20 = zc
    m21 = zc
    m22 = gl[:, 2]
    det = (m00 * (m11 * m22 - m12 * m21)
           - m01 * (m10 * m22 - m12 * m20)
           + m02 * (m10 * m21 - m11 * m20))
    rdet = 1.0 / det
    i00 = (m11 * m22 - m12 * m21) * rdet
    i01 = (m02 * m21 - m01 * m22) * rdet
    i02 = (m01 * m12 - m02 * m11) * rdet
    i10 = (m12 * m20 - m10 * m22) * rdet
    i11 = (m00 * m22 - m02 * m20) * rdet
    i12 = (m02 * m10 - m00 * m12) * rdet
    i20 = (m10 * m21 - m11 * m20) * rdet
    i21 = (m01 * m20 - m00 * m21) * rdet
    i22 = (m00 * m11 - m01 * m10) * rdet
    cols = [m00, m01, m02, m10, m11, m12, m20, m21, m22,
            i00, i01, i02, i10, i11, i12, i20, i21, i22,
            sig, tsig, m1, j1.astype(_f32), zc, zc]
    crys_o[...] = jnp.stack(cols, axis=1)
    zw_o[...] = z @ wd1[...] + bd1[...]

    @pl.when(pl.program_id(0) == 0)
    def _():
        tw_o[...] = temb[...] @ wd1[...]


def _tca(hidden, eps, W_mu, b_mu, W_var, b_var, Wn1, bn1, Wn2p, bn2p,
         Wl1, bl1, Wl2p, bl2p, Wc1, bc1, Wc2, bc2, Wd1, bd1, type_emb,
         nat2, glu, gau, nl2, tnl2, sigt, tsigt, sm6, ss6):
    row = lambda i: (i, 0)
    fix = lambda i: (0, 0)
    mk = lambda shp: jax.ShapeDtypeStruct(shp, _f32)
    in_specs = (
        [pl.BlockSpec((BB, D), row), pl.BlockSpec((BB, D), row)]
        + [pl.BlockSpec(w.shape, fix) for w in
           (W_mu, b_mu, W_var, b_var, Wn1, bn1, Wn2p, bn2p,
            Wl1, bl1, Wl2p, bl2p, Wc1, bc1, Wc2, bc2, Wd1, bd1, type_emb)]
        + [pl.BlockSpec((BB, 1), row), pl.BlockSpec((BB, 3), row),
           pl.BlockSpec((BB, 3), row), pl.BlockSpec((BB, 1), row),
           pl.BlockSpec((BB, 1), row),
           pl.BlockSpec((1, 64), fix), pl.BlockSpec((1, 64), fix),
           pl.BlockSpec((1, 6), fix), pl.BlockSpec((1, 6), fix)]
    )
    out_shape = [mk((B, D)), mk((B, D)), mk((B, 104)), mk((B, 128)), mk((B, 128)),
                 mk((B, 24)), mk((B, 128)), mk((B, 16)), mk((B, 1)), mk((B, 1)),
                 mk((A, D))]
    out_specs = [pl.BlockSpec((BB, D), row), pl.BlockSpec((BB, D), row),
                 pl.BlockSpec((BB, 104), row), pl.BlockSpec((BB, 128), row),
                 pl.BlockSpec((BB, 128), row), pl.BlockSpec((BB, 24), row),
                 pl.BlockSpec((BB, 128), row), pl.BlockSpec((BB, 16), row),
                 pl.BlockSpec((BB, 1), row), pl.BlockSpec((BB, 1), row),
                 pl.BlockSpec((A, D), fix)]
    return pl.pallas_call(
        _tca_body, grid=(B // BB,), in_specs=in_specs,
        out_specs=out_specs, out_shape=out_shape,
    )(hidden, eps, W_mu, b_mu, W_var, b_var, Wn1, bn1, Wn2p, bn2p,
      Wl1, bl1, Wl2p, bl2p, Wc1, bc1, Wc2, bc2, Wd1, bd1, type_emb,
      nat2, glu, gau, nl2, tnl2, sigt, tsigt, sm6, ss6)


def _sc1_body(bi_h, at_h, comp_h, logpf_h, ptsf_h, sigc_h, tsigc_h, m1c_h,
              j1c_h, zw_h, tw_h, zer2_h, zer1_h,
              compo_h, h0_h, sigpa_h, tsigpa_h, segp_h, cntp_h, cep_h,
              biv, atv, idx2v, rtv, pav, lcev, m1v, j1v, sigv, tsigv,
              onesv, cenv, compv, zwv, twv, shd, shc, she, sem):
    cid = lax.axis_index("c")
    sid = lax.axis_index("s")
    wid = cid * NS + sid
    pltpu.sync_copy(zer2_h.at[pl.ds(sid * 256, 256)], shd.at[pl.ds(sid * 256, 256)])
    pltpu.sync_copy(zer1_h.at[pl.ds(sid * 256, 256)], shc.at[pl.ds(sid * 256, 256)])
    pltpu.sync_copy(zer1_h.at[pl.ds(sid * 256, 256)], she.at[pl.ds(sid * 256, 256)])
    plsc.subcore_barrier()
    for g in range(CH1 // 16):
        onesv[pl.ds(g * 16, 16)] = jnp.full((16,), 1.0, _f32)

    def chunk(i, carry):
        base = wid * APW + i * CH1
        c1 = pltpu.async_copy(bi_h.at[pl.ds(base, CH1)], biv, sem)
        c2 = pltpu.async_copy(at_h.at[pl.ds(base, CH1)], atv, sem)
        c1.wait(); c2.wait()
        for g in range(CH1 // 16):
            sl = pl.ds(g * 16, 16)
            idx2v[sl] = biv[sl] * 128 + atv[sl]
        g1 = pltpu.async_copy(sigc_h.at[biv], sigv, sem)
        g2 = pltpu.async_copy(tsigc_h.at[biv], tsigv, sem)
        g3 = pltpu.async_copy(m1c_h.at[biv], m1v, sem)
        g4 = pltpu.async_copy(j1c_h.at[biv], j1v, sem)
        g5 = pltpu.async_copy(ptsf_h.at[idx2v], pav, sem)
        g6 = pltpu.async_copy(logpf_h.at[idx2v], lcev, sem)
        g7 = pltpu.async_copy(zw_h.at[biv], zwv, sem)
        g8 = pltpu.async_copy(comp_h.at[biv], compv, sem)
        g1.wait(); g2.wait(); g3.wait(); g4.wait()
        g5.wait(); g6.wait(); g7.wait(); g8.wait()
        for g in range(CH1 // 16):
            sl = pl.ds(g * 16, 16)
            a16 = atv[sl]
            j1i = j1v[sl].astype(_i32)
            m116 = m1v[sl]
            pa1 = pav[sl] + 1.0
            rtv[sl] = jnp.where(
                pa1 > m116, a16,
                jnp.where(pa1 == m116, jnp.minimum(a16, j1i), j1i))
            cenv[sl] = 0.0 - lcev[sl]
        gt = pltpu.async_copy(tw_h.at[rtv], twv, sem)
        gt.wait()

        def hrow(r, c2):
            for cidx in range(D // 16):
                sl = pl.ds(cidx * 16, 16)
                zwv[r, sl] = jnp.maximum(zwv[r, sl] + twv[r, sl], 0.0)
            return c2
        lax.fori_loop(0, CH1, hrow, 0)
        o1 = pltpu.async_copy(compv, compo_h.at[pl.ds(base, CH1)], sem)
        o3 = pltpu.async_copy(zwv, h0_h.at[pl.ds(base, CH1)], sem)
        o4 = pltpu.async_copy(sigv, sigpa_h.at[pl.ds(base, CH1)], sem)
        o5 = pltpu.async_copy(tsigv, tsigpa_h.at[pl.ds(base, CH1)], sem)
        pltpu.sync_copy(zwv, shd.at[biv], add=True)
        pltpu.sync_copy(onesv, shc.at[biv], add=True)
        pltpu.sync_copy(cenv, she.at[biv], add=True)
        o1.wait(); o3.wait(); o4.wait(); o5.wait()
        return carry
    lax.fori_loop(0, NCH1, chunk, 0)
    plsc.subcore_barrier()
    off = cid * B + sid * 256
    pltpu.sync_copy(shd.at[pl.ds(sid * 256, 256)], segp_h.at[pl.ds(off, 256)])
    pltpu.sync_copy(shc.at[pl.ds(sid * 256, 256)], cntp_h.at[pl.ds(off, 256)])
    pltpu.sync_copy(she.at[pl.ds(sid * 256, 256)], cep_h.at[pl.ds(off, 256)])


def _sc1(bi, at, comp100, logpf, ptsf, sigc, tsigc, m1c, j1c, zw, tw,
         zer2, zer1):
    mesh = plsc.VectorSubcoreMesh(core_axis_name="c", subcore_axis_name="s",
                                  num_cores=NC, num_subcores=NS)
    mk = lambda shp, dt=_f32: jax.ShapeDtypeStruct(shp, dt)
    f = pl.kernel(
        _sc1_body,
        out_type=(mk((N, 104)), mk((N, D)), mk((N,)), mk((N,)),
                  mk((2 * B, D)), mk((2 * B,)), mk((2 * B,))),
        mesh=mesh,
        scratch_types=(
            pltpu.VMEM((CH1,), _i32), pltpu.VMEM((CH1,), _i32),
            pltpu.VMEM((CH1,), _i32), pltpu.VMEM((CH1,), _i32),
            pltpu.VMEM((CH1,), _f32), pltpu.VMEM((CH1,), _f32),
            pltpu.VMEM((CH1,), _f32), pltpu.VMEM((CH1,), _f32),
            pltpu.VMEM((CH1,), _f32), pltpu.VMEM((CH1,), _f32),
            pltpu.VMEM((CH1,), _f32), pltpu.VMEM((CH1,), _f32),
            pltpu.VMEM((CH1, 104), _f32), pltpu.VMEM((CH1, D), _f32),
            pltpu.VMEM((CH1, D), _f32),
            pltpu.VMEM_SHARED((B, D), _f32), pltpu.VMEM_SHARED((B,), _f32),
            pltpu.VMEM_SHARED((B,), _f32),
            pltpu.SemaphoreType.DMA,
        ),
        compiler_params=pltpu.CompilerParams(use_tc_tiling_on_sc=False),
    )
    return f(bi, at, comp100, logpf, ptsf, sigc, tsigc, m1c, j1c, zw, tw,
             zer2, zer1)


def _tcb_body(s0, s1, c0, c1, e0, e1, wap, aggt_o, cnt_o, ces_o):
    cnt = jnp.maximum(c0[0] + c1[0], 1.0)
    agg = (s0[0] + s1[0]) / cnt
    aggt_o[...] = agg @ wap[...]
    cnt_o[...] = cnt
    ces_o[...] = e0[0] + e1[0]


def _tcb(segp3, cntp3, cep3, WaP):
    half = lambda h: (lambda i: (h, i, 0))
    row = lambda i: (i, 0)
    fix = lambda i: (0, 0)
    mk = lambda shp: jax.ShapeDtypeStruct(shp, _f32)
    return pl.pallas_call(
        _tcb_body, grid=(B // BBM,),
        in_specs=[pl.BlockSpec((1, BBM, D), half(0)), pl.BlockSpec((1, BBM, D), half(1)),
                  pl.BlockSpec((1, BBM, 1), half(0)), pl.BlockSpec((1, BBM, 1), half(1)),
                  pl.BlockSpec((1, BBM, 1), half(0)), pl.BlockSpec((1, BBM, 1), half(1)),
                  pl.BlockSpec((D, 128), fix)],
        out_specs=[pl.BlockSpec((BBM, 128), row), pl.BlockSpec((BBM, 1), row),
                   pl.BlockSpec((BBM, 1), row)],
        out_shape=[mk((B, 128)), mk((B, 1)), mk((B, 1))],
    )(segp3, segp3, cntp3, cntp3, cep3, cep3, WaP)


def _sc2_body(bi_h, aggt_h, crys_h, out_h, cro_h, biv, rv, cv, sem):
    cid = lax.axis_index("c")
    sid = lax.axis_index("s")
    wid = cid * NS + sid

    def chunk(i, carry):
        base = wid * APW + i * CH
        pltpu.sync_copy(bi_h.at[pl.ds(base, CH)], biv)
        d1 = pltpu.async_copy(aggt_h.at[biv], rv, sem)
        d2 = pltpu.async_copy(crys_h.at[biv], cv, sem)
        d1.wait(); d2.wait()
        o1 = pltpu.async_copy(rv, out_h.at[pl.ds(base, CH)], sem)
        o2 = pltpu.async_copy(cv, cro_h.at[pl.ds(base, CH)], sem)
        o1.wait(); o2.wait()
        return carry
    lax.fori_loop(0, NCHUNK, chunk, 0)


def _sc2(bi, aggt, crys24):
    mesh = plsc.VectorSubcoreMesh(core_axis_name="c", subcore_axis_name="s",
                                  num_cores=NC, num_subcores=NS)
    f = pl.kernel(
        _sc2_body,
        out_type=(jax.ShapeDtypeStruct((N, 128), _f32),
                  jax.ShapeDtypeStruct((N, 24), _f32)),
        mesh=mesh,
        scratch_types=(pltpu.VMEM((CH,), _i32), pltpu.VMEM((CH, 128), _f32),
                       pltpu.VMEM((CH, 24), _f32), pltpu.SemaphoreType.DMA),
        compiler_params=pltpu.CompilerParams(use_tc_tiling_on_sc=False),
    )
    return f(bi, aggt, crys24)


def _tcc_body(h0, aggpa, sig, tsig, noi, at2, fr, crp, wap, ba, bd2,
              pats_o, pd_o, nf_o, cet_o, csq_o):
    u = h0[...] @ wap[...] + aggpa[...]
    pats = u[:, :A] + ba[...]
    pats_o[...] = pats
    pd = u[:, A:A + 3] + bd2[...]
    pd_o[...] = pd
    mx = jnp.max(pats, axis=1, keepdims=True)
    e = jnp.exp(pats - mx)
    lse = jnp.log(jnp.sum(e, axis=1, keepdims=True))
    i100 = lax.broadcasted_iota(_i32, pats.shape, 1)
    lat = jnp.sum(jnp.where(i100 == at2[...], pats, 0.0), axis=1, keepdims=True)
    cet_o[...] = (mx + lse - lat) / tsig[...]
    noi_ = noi[...]
    cr = pd / sig[...] + noi_
    csq_o[...] = jnp.sum(cr * cr, axis=1, keepdims=True)
    frb = fr[...]
    crpb = crp[...]
    col = lambda m, k: m[:, k:k + 1]
    fx, fy, fz = col(frb, 0), col(frb, 1), col(frb, 2)
    sg = sig[...]
    l_ = [col(crpb, k) for k in range(9)]
    iv = [col(crpb, 9 + k) for k in range(9)]
    cx = fx * l_[0] + fy * l_[3] + fz * l_[6] + col(noi_, 0) * sg
    cy = fx * l_[1] + fy * l_[4] + fz * l_[7] + col(noi_, 1) * sg
    cz = fx * l_[2] + fy * l_[5] + fz * l_[8] + col(noi_, 2) * sg
    ux = cx * iv[0] + cy * iv[3] + cz * iv[6]
    uy = cx * iv[1] + cy * iv[4] + cz * iv[7]
    uz = cx * iv[2] + cy * iv[5] + cz * iv[8]
    uu = jnp.concatenate([ux, uy, uz], axis=1)
    r = lax.rem(uu, jnp.ones_like(uu))
    nf_o[...] = jnp.where((r != 0.0) & (r < 0.0), r + 1.0, r)


def _tcc(h0, aggpa, sigpa2, tsigpa2, cart_noise, at2, frac, cryspa,
         WaP, ba2, bd22):
    row = lambda i: (i, 0)
    fix = lambda i: (0, 0)
    mk = lambda shp: jax.ShapeDtypeStruct(shp, _f32)
    return pl.pallas_call(
        _tcc_body, grid=(N // NB,),
        in_specs=[pl.BlockSpec((NB, D), row), pl.BlockSpec((NB, 128), row),
                  pl.BlockSpec((NB, 1), row), pl.BlockSpec((NB, 1), row),
                  pl.BlockSpec((NB, 3), row), pl.BlockSpec((NB, 1), row),
                  pl.BlockSpec((NB, 3), row), pl.BlockSpec((NB, 24), row),
                  pl.BlockSpec((D, 128), fix), pl.BlockSpec((1, A), fix),
                  pl.BlockSpec((1, 3), fix)],
        out_specs=[pl.BlockSpec((NB, A), row), pl.BlockSpec((NB, 3), row),
                   pl.BlockSpec((NB, 3), row),
                   pl.BlockSpec((NB, 1), row), pl.BlockSpec((NB, 1), row)],
        out_shape=[mk((N, A)), mk((N, 3)), mk((N, 3)), mk((N, 1)), mk((N, 1))],
    )(h0, aggpa, sigpa2, tsigpa2, cart_noise, at2, frac, cryspa,
      WaP, ba2, bd22)


def _sc3_body(bi_h, cet_h, csq_h, zer1_h, cetp_h, csqp_h,
              biv, v1, v2, s1, s2, sem):
    cid = lax.axis_index("c")
    sid = lax.axis_index("s")
    wid = cid * NS + sid
    pltpu.sync_copy(zer1_h.at[pl.ds(sid * 256, 256)], s1.at[pl.ds(sid * 256, 256)])
    pltpu.sync_copy(zer1_h.at[pl.ds(sid * 256, 256)], s2.at[pl.ds(sid * 256, 256)])
    plsc.subcore_barrier()

    def chunk(i, carry):
        base = wid * APW + i * CH
        c1 = pltpu.async_copy(bi_h.at[pl.ds(base, CH)], biv, sem)
        c2 = pltpu.async_copy(cet_h.at[pl.ds(base, CH)], v1, sem)
        c3 = pltpu.async_copy(csq_h.at[pl.ds(base, CH)], v2, sem)
        c1.wait(); c2.wait(); c3.wait()
        pltpu.sync_copy(v1, s1.at[biv], add=True)
        pltpu.sync_copy(v2, s2.at[biv], add=True)
        return carry
    lax.fori_loop(0, NCHUNK, chunk, 0)
    plsc.subcore_barrier()
    off = cid * B + sid * 256
    pltpu.sync_copy(s1.at[pl.ds(sid * 256, 256)], cetp_h.at[pl.ds(off, 256)])
    pltpu.sync_copy(s2.at[pl.ds(sid * 256, 256)], csqp_h.at[pl.ds(off, 256)])


def _sc3(bi, cet, csq, zer1):
    mesh = plsc.VectorSubcoreMesh(core_axis_name="c", subcore_axis_name="s",
                                  num_cores=NC, num_subcores=NS)
    mk = lambda shp: jax.ShapeDtypeStruct(shp, _f32)
    f = pl.kernel(
        _sc3_body,
        out_type=(mk((2 * B,)), mk((2 * B,))),
        mesh=mesh,
        scratch_types=(pltpu.VMEM((CH,), _i32), pltpu.VMEM((CH,), _f32),
                       pltpu.VMEM((CH,), _f32),
                       pltpu.VMEM_SHARED((B,), _f32), pltpu.VMEM_SHARED((B,), _f32),
                       pltpu.SemaphoreType.DMA),
        compiler_params=pltpu.CompilerParams(use_tc_tiling_on_sc=False),
    )
    return f(bi, cet, csq, zer1)


def _tcf_body(pnum, nat2, lmse, kldr, ces, cnt, cetp, csqp, tot_o):
    L = pnum[:, :21]
    mx = jnp.max(L, axis=1, keepdims=True)
    lse = jnp.log(jnp.sum(jnp.exp(L - mx), axis=1, keepdims=True))
    na = nat2[...] + 1
    i21 = lax.broadcasted_iota(_i32, L.shape, 1)
    lat = jnp.sum(jnp.where(i21 == na, L, 0.0), axis=1, keepdims=True)
    na_loss = jnp.sum(mx + lse - lat) / B
    lattice_loss = jnp.sum(lmse[...]) / (B * 6.0)
    cnt_ = cnt[...]
    comp_loss = jnp.sum(ces[...] / cnt_) / B
    cets = cetp[0:B, :] + cetp[B:2 * B, :]
    csqs = csqp[0:B, :] + csqp[B:2 * B, :]
    type_loss = jnp.sum(cets / cnt_) / B
    coord_loss = jnp.sum(csqs / cnt_) / B
    kld = jnp.sum(kldr[...]) / B
    tot = (na_loss + 10.0 * lattice_loss + comp_loss
           + 10.0 * coord_loss + type_loss + 0.01 * kld)
    tot_o[...] = jnp.reshape(tot, (1, 1))


def _tcf(pnum_p, nat2, lmse, kldr, ces, cnt, cetp2, csqp2):
    return pl.pallas_call(
        _tcf_body,
        out_shape=jax.ShapeDtypeStruct((1, 1), _f32),
    )(pnum_p, nat2, lmse, kldr, ces, cnt, cetp2, csqp2)


def kernel(hidden, eps, frac_coords, gt_lengths_u, gt_angles_u, cart_noise,
           W_mu, b_mu, W_var, b_var, Wn1, bn1, Wn2, bn2, Wl1, bl1, Wl2, bl2,
           Wc1, bc1, Wc2, bc2, type_emb, Wd1, bd1, Wd2, bd2, Wa, ba,
           scaler_mean, scaler_std, sigmas, type_sigmas, batch_idx,
           atom_types, num_atoms_t, noise_level, type_noise_level):
    f32 = _f32
    r2 = lambda v: v.reshape(1, -1).astype(f32)
    Wn2p = jnp.pad(Wn2, ((0, 0), (0, 128 - Wn2.shape[1])))
    bn2p = jnp.pad(r2(bn2), ((0, 0), (0, 128 - bn2.shape[0])))
    Wl2p = jnp.pad(Wl2, ((0, 0), (0, 128 - Wl2.shape[1])))
    bl2p = jnp.pad(r2(bl2), ((0, 0), (0, 128 - bl2.shape[0])))
    sigt = jnp.pad(r2(sigmas), ((0, 0), (0, 64 - NNL)))
    tsigt = jnp.pad(r2(type_sigmas), ((0, 0), (0, 64 - NNL)))
    (z, zw, comp100, logp_p, pts_p, crys24, pnum_p, pla_pack, lmse, kldr,
     tw) = _tca(
        hidden, eps, W_mu, r2(b_mu), W_var, r2(b_var), Wn1, r2(bn1), Wn2p,
        bn2p, Wl1, r2(bl1), Wl2p, bl2p, Wc1, r2(bc1), Wc2, r2(bc2), Wd1,
        r2(bd1), type_emb,
        num_atoms_t.reshape(B, 1), gt_lengths_u, gt_angles_u,
        noise_level.reshape(B, 1), type_noise_level.reshape(B, 1),
        sigt, tsigt, r2(scaler_mean), r2(scaler_std))

    bi = batch_idx.astype(_i32)
    at = atom_types.astype(_i32)
    zer2 = jnp.zeros((B, D), f32)
    zer1 = jnp.zeros((B,), f32)
    sigc = crys24[:, 18]
    tsigc = crys24[:, 19]
    m1c = crys24[:, 20]
    j1c = crys24[:, 21]
    (comp104, h0, sigpa, tsigpa, segp, cntp, cep) = _sc1(
        bi, at, comp100, logp_p.reshape(B * 128),
        pts_p.reshape(B * 128), sigc, tsigc, m1c, j1c, zw, tw, zer2, zer1)

    WaP = jnp.concatenate([Wa, Wd2, jnp.zeros((D, 128 - A - 3), f32)], axis=1)
    aggt, cnt, ces = _tcb(segp.reshape(2, B, D), cntp.reshape(2, B, 1),
                          cep.reshape(2, B, 1), WaP)
    aggpa, cryspa = _sc2(bi, aggt, crys24)
    pats, pred_diff, noisy_frac, cet, csq = _tcc(
        h0, aggpa, sigpa.reshape(N, 1), tsigpa.reshape(N, 1), cart_noise,
        at.reshape(N, 1), frac_coords, cryspa, WaP, r2(ba), r2(bd2))
    cetp, csqp = _sc3(bi, cet.reshape(N), csq.reshape(N), zer1)
    tot = _tcf(pnum_p, num_atoms_t.reshape(B, 1), lmse, kldr, ces, cnt,
               cetp.reshape(2 * B, 1), csqp.reshape(2 * B, 1))

    total = tot[0, 0]
    pred_num_atoms = pnum_p[:, :21]
    pred_la = pla_pack[:, :6]
    pred_lengths = pla_pack[:, 6:9]
    pred_angles = pla_pack[:, 9:12]
    comp = comp104[:, :A]
    return (total, pred_num_atoms, pred_la, pred_lengths, pred_angles, comp,
            pred_diff, pats, noisy_frac, z)
